# baseline (speedup 1.0000x reference)
import functools
import jax
import jax.numpy as jnp
from jax import lax
from jax.experimental import pallas as pl
from jax.experimental.pallas import tpu as pltpu
from jax.experimental.pallas import tpu_sc as plsc

N, S, D = 4, 4096, 1024
NW = 32
POS_PER_W = S // NW
C = 16
PCHUNKS = POS_PER_W // C
STEPS = PCHUNKS * N

_mesh = plsc.VectorSubcoreMesh(core_axis_name="c", subcore_axis_name="s")


@functools.partial(
    pl.kernel,
    mesh=_mesh,
    out_type=jax.ShapeDtypeStruct((N * S, D), jnp.float32),
    scratch_types=[
        pltpu.VMEM((2, C, D), jnp.float32),
        pltpu.VMEM((2, C, D), jnp.float32),
        pltpu.VMEM((2, C, D), jnp.float32),
        pltpu.SemaphoreType.DMA((2,)),
        pltpu.SemaphoreType.DMA((2,)),
        pltpu.SemaphoreType.DMA((2,)),
    ],
    compiler_params=pltpu.CompilerParams(use_tc_tiling_on_sc=True),
)
def _sc_add(x_hbm, enc_hbm, out_hbm, xv, pv, ov, sx, sp, so):
    wid = lax.axis_index("s") * 2 + lax.axis_index("c")
    prow0 = wid * POS_PER_W

    def xrow(g, n):
        return n * S + prow0 + g * C

    def start_x(g, n, b):
        pltpu.async_copy(x_hbm.at[pl.ds(xrow(g, n), C), :], xv.at[b], sx.at[b])

    def wait_x(g, n, b):
        pltpu.make_async_copy(
            x_hbm.at[pl.ds(xrow(g, n), C), :], xv.at[b], sx.at[b]).wait()

    def start_pe(g, b):
        pltpu.async_copy(
            enc_hbm.at[pl.ds(prow0 + g * C, C), :], pv.at[b], sp.at[b])

    def wait_pe(g, b):
        pltpu.make_async_copy(
            enc_hbm.at[pl.ds(prow0 + g * C, C), :], pv.at[b], sp.at[b]).wait()

    def start_out(g, n, b):
        pltpu.async_copy(
            ov.at[b], out_hbm.at[pl.ds(xrow(g, n), C), :], so.at[b])

    def wait_out(g, n, b):
        pltpu.make_async_copy(
            ov.at[b], out_hbm.at[pl.ds(xrow(g, n), C), :], so.at[b]).wait()

    start_pe(0, 0)
    start_pe(1, 1)
    start_x(0, 0, 0)
    start_x(0, 1, 1)

    def step2(g2, carry):
        for gp in range(2):
            g = g2 * 2 + gp
            gb = gp
            for n in range(N):
                i = g * N + n
                ib = n % 2
                if n == 0:
                    wait_pe(g, gb)
                wait_x(g, n, ib)

                @pl.when(i >= 2)
                def _drain():
                    g5, n5 = (g, n - 2) if n >= 2 else (g - 1, n + 2)
                    wait_out(g5, n5, ib)

                def body(r, c2):
                    for j in range(D // 16):
                        s = j * 16
                        ov[ib, r, pl.ds(s, 16)] = (
                            xv[ib, r, pl.ds(s, 16)] + pv[gb, r, pl.ds(s, 16)])
                    return c2

                lax.fori_loop(0, C, body, 0)
                start_out(g, n, ib)

                @pl.when(i + 2 < STEPS)
                def _prefetch():
                    g3, n3 = (g, n + 2) if n + 2 < N else (g + 1, n - 2)
                    start_x(g3, n3, ib)

                if n == N - 1:
                    @pl.when(g + 2 < PCHUNKS)
                    def _pe_pref():
                        start_pe(g + 2, gb)
        return carry

    lax.fori_loop(0, PCHUNKS // 2, step2, 0)
    wait_out(PCHUNKS - 1, N - 2, 0)
    wait_out(PCHUNKS - 1, N - 1, 1)


def kernel(x, encoding):
    out = _sc_add(x.reshape(N * S, D), encoding)
    return out.reshape(x.shape)


# speedup vs baseline: 1.5748x; 1.5748x over previous
import functools
import jax
import jax.numpy as jnp
from jax import lax
from jax.experimental import pallas as pl
from jax.experimental.pallas import tpu as pltpu
from jax.experimental.pallas import tpu_sc as plsc

N, S, D = 4, 4096, 1024
NW = 32
POS_PER_W = S // NW
C = 8
PCHUNKS = POS_PER_W // C

_mesh = plsc.VectorSubcoreMesh(core_axis_name="c", subcore_axis_name="s")


@functools.partial(
    pl.kernel,
    mesh=_mesh,
    out_type=jax.ShapeDtypeStruct((N * S, D), jnp.float32),
    scratch_types=[
        pltpu.VMEM((N, 2, C, D), jnp.float32),
        pltpu.VMEM((2, C, D), jnp.float32),
        pltpu.SemaphoreType.DMA((N, 2)),
        pltpu.SemaphoreType.DMA((2,)),
        pltpu.SemaphoreType.DMA((N, 2)),
    ],
    compiler_params=pltpu.CompilerParams(use_tc_tiling_on_sc=True),
)
def _sc_add(x_hbm, enc_hbm, out_hbm, xv, pv, sx, sp, so):
    wid = lax.axis_index("s") * 2 + lax.axis_index("c")
    prow0 = wid * POS_PER_W

    def xrow(g, n):
        return n * S + prow0 + g * C

    def start_x(g, n, p):
        pltpu.async_copy(
            x_hbm.at[pl.ds(xrow(g, n), C), :], xv.at[n, p], sx.at[n, p])

    def wait_x(g, n, p):
        pltpu.make_async_copy(
            x_hbm.at[pl.ds(xrow(g, n), C), :], xv.at[n, p], sx.at[n, p]).wait()

    def start_pe(g, b):
        pltpu.async_copy(
            enc_hbm.at[pl.ds(prow0 + g * C, C), :], pv.at[b], sp.at[b])

    def wait_pe(g, b):
        pltpu.make_async_copy(
            enc_hbm.at[pl.ds(prow0 + g * C, C), :], pv.at[b], sp.at[b]).wait()

    def start_out(g, n, p):
        pltpu.async_copy(
            xv.at[n, p], out_hbm.at[pl.ds(xrow(g, n), C), :], so.at[n, p])

    def wait_out(g, n, p):
        pltpu.make_async_copy(
            xv.at[n, p], out_hbm.at[pl.ds(xrow(g, n), C), :],
            so.at[n, p]).wait()

    start_pe(0, 0)
    start_pe(1, 1)
    for n in range(N):
        start_x(0, n, 0)

    def step2(g2, carry):
        for gp in range(2):
            g = g2 * 2 + gp
            for n in range(N):
                if n == 0:
                    wait_pe(g, gp)
                wait_x(g, n, gp)

                def body(r, c2):
                    for j in range(D // 16):
                        s = j * 16
                        plsc.addupdate(
                            xv.at[n, gp, r, pl.ds(s, 16)],
                            pv[gp, r, pl.ds(s, 16)])
                    return c2

                lax.fori_loop(0, C, body, 0)
                start_out(g, n, gp)

                @pl.when(g + 1 < PCHUNKS)
                def _prefetch():
                    @pl.when(g >= 1)
                    def _drain():
                        wait_out(g - 1, n, 1 - gp)
                    start_x(g + 1, n, 1 - gp)

                if n == N - 1:
                    @pl.when(g + 2 < PCHUNKS)
                    def _pe_pref():
                        start_pe(g + 2, gp)
        return carry

    lax.fori_loop(0, PCHUNKS // 2, step2, 0)
    for n in range(N):
        wait_out(PCHUNKS - 1, n, (PCHUNKS - 1) % 2)


def kernel(x, encoding):
    out = _sc_add(x.reshape(N * S, D), encoding)
    return out.reshape(x.shape)
